# baseline (speedup 1.0000x reference)
import functools

import jax
import jax.numpy as jnp
from jax import lax
from jax.experimental import pallas as pl
from jax.experimental.pallas import tpu as pltpu
from jax.experimental.pallas import tpu_sc as plsc

B, N = 2, 100000
H, WG = 256, 256
FD = 128
EPS = 1e-5
TOT = B * N
CELLS = H * WG
BLK = 8000
NBLK = TOT // BLK

NT = 16
SHARD = 6256
SHARD_LAST = N - 15 * SHARD
PAD_PTS = 192
UROWS = 512
NREG = 8
WST = 512
CHQ = 2048
QCAP = N + 256 * 256 + CHQ
TRASH = 8 << 27
PIDM = (1 << 18) - 1


def _mlp_body(pts_ref, a1_ref, c1_ref, a2_ref, c2_ref, a3_ref, c3_ref,
              feat_ref, idx_ref):
    x = pts_ref[...]
    h = jnp.maximum(jnp.dot(x, a1_ref[...], preferred_element_type=jnp.float32)
                    + c1_ref[...], 0.0)
    h = jnp.maximum(jnp.dot(h, a2_ref[...], preferred_element_type=jnp.float32)
                    + c2_ref[...], 0.0)
    h = jnp.maximum(jnp.dot(h, a3_ref[...], preferred_element_type=jnp.float32)
                    + c3_ref[...], 0.0)
    feat_ref[...] = h

    bid = pl.program_id(0)
    row = bid * BLK + jax.lax.broadcasted_iota(jnp.int32, (BLK, 1), 0)
    b = row // N
    xx = x[:, 0:1]
    yy = x[:, 1:2]
    xn = (xx + 50.0) * 0.01
    yn = (yy + 50.0) * 0.01
    valid = (xn >= 0) & (xn <= 1) & (yn >= 0) & (yn <= 1)
    gx = jnp.clip((xn * (WG - 1)).astype(jnp.int32), 0, WG - 1)
    gy = jnp.clip((yn * (H - 1)).astype(jnp.int32), 0, H - 1)
    flat = b * CELLS + gy * WG + gx
    flat = jnp.where(valid, flat, B * CELLS)
    idx_ref[...] = flat


def _mlp(pts, a1, c1, a2, c2, a3, c3):
    return pl.pallas_call(
        _mlp_body,
        grid=(NBLK,),
        in_specs=[
            pl.BlockSpec((BLK, 4), lambda i: (i, 0)),
            pl.BlockSpec((4, 64), lambda i: (0, 0)),
            pl.BlockSpec((1, 64), lambda i: (0, 0)),
            pl.BlockSpec((64, 128), lambda i: (0, 0)),
            pl.BlockSpec((1, 128), lambda i: (0, 0)),
            pl.BlockSpec((128, 128), lambda i: (0, 0)),
            pl.BlockSpec((1, 128), lambda i: (0, 0)),
        ],
        out_specs=[
            pl.BlockSpec((BLK, 128), lambda i: (i, 0)),
            pl.BlockSpec((BLK, 1), lambda i: (i, 0)),
        ],
        out_shape=[
            jax.ShapeDtypeStruct((TOT, 128), jnp.float32),
            jax.ShapeDtypeStruct((TOT, 1), jnp.int32),
        ],
    )(pts, a1, c1, a2, c2, a3, c3)


_sc_mesh = plsc.VectorSubcoreMesh(core_axis_name="c", subcore_axis_name="s")


@functools.partial(
    pl.kernel,
    out_type=jax.ShapeDtypeStruct((B, FD, CELLS), jnp.float32),
    mesh=_sc_mesh,
    scratch_types=[
        pltpu.VMEM((FD, WST), jnp.float32),
        pltpu.VMEM((SHARD,), jnp.int32),
        pltpu.VMEM((SHARD + 272,), jnp.int32),
        pltpu.VMEM((SHARD,), jnp.int32),
        pltpu.VMEM((SHARD,), jnp.int32),
        pltpu.VMEM((CHQ,), jnp.int32),
        pltpu.VMEM((16, FD), jnp.float32),
        pltpu.VMEM((CHQ + 32,), jnp.int32),
        pltpu.VMEM((CHQ + 32,), jnp.int32),
        pltpu.VMEM((16, 16), jnp.int32),
        pltpu.VMEM((16,), jnp.int32),
        pltpu.VMEM((16,), jnp.int32),
        pltpu.VMEM((256,), jnp.int32),
        pltpu.VMEM_SHARED((16, 16), jnp.int32),
        pltpu.VMEM_SHARED((QCAP,), jnp.int32),
        pltpu.VMEM_SHARED((16, WST), jnp.float32),
        pltpu.SemaphoreType.DMA,
    ],
    compiler_params=pltpu.CompilerParams(needs_layout_passes=False),
)
def _scatter_sc(feat_hbm, idx_hbm, out_hbm, win, idxbuf, outseg,
                packbuf, ownbuf, qstage, fstage, rlist, plist, gg, svec,
                rbuf, hist, cnt_sh, qbuf, zb_sh, sem):
    c = lax.axis_index("c")
    s = lax.axis_index("s")
    iota = lax.broadcasted_iota(jnp.int32, (16,), 0)
    zeros16 = jnp.zeros((16,), jnp.int32)
    zerosf16 = jnp.zeros((16,), jnp.float32)
    ones16 = jnp.ones((16,), jnp.int32)
    fvecs = [iota + fc * 16 for fc in range(8)]

    @pl.when(s == 0)
    def _init_zb():
        for row in range(16):
            def zz_body(k, carry, row=row):
                win[row, pl.ds(k * 16, 16)] = jnp.zeros((16,), jnp.float32)
                return carry
            lax.fori_loop(0, WST // 16, zz_body, 0)
        pltpu.sync_copy(win.at[pl.ds(0, 16), :], zb_sh)

    nmine = jnp.where(s < NT - 1, SHARD, SHARD_LAST)
    n16 = nmine // 16
    base = c * N + s * SHARD

    pltpu.sync_copy(idx_hbm.at[pl.ds(pl.multiple_of(base, 16), SHARD)], idxbuf)

    for k in range(16):
        hist[pl.ds(k * 16, 16)] = zeros16

    def p1_body(i, carry):
        v = idxbuf[pl.ds(i * 16, 16)]
        local = jnp.minimum(v - c * CELLS, CELLS)
        unit = local >> 9
        owner = unit & 15
        region = unit >> 4
        r = local & 511
        pid = s * SHARD + i * 16 + iota
        pk = (region << 27) | (r << 18) | pid
        packbuf[pl.ds(i * 16, 16)] = pk
        ownbuf[pl.ds(i * 16, 16)] = owner
        plsc.addupdate_scatter(hist, [owner * 16 + iota], ones16)
        return carry

    lax.fori_loop(0, n16, p1_body, 0)

    cvec = zeros16
    for d in range(16):
        cs = jnp.sum(hist[pl.ds(d * 16, 16)])
        cvec = jnp.where(iota == d, zeros16 + cs, cvec)
    svec[...] = cvec
    pltpu.sync_copy(svec, cnt_sh.at[s])
    plsc.subcore_barrier()

    pltpu.sync_copy(cnt_sh, gg)
    totp = zeros16
    pmine = zeros16
    svec_b = zeros16 + s
    for sp in range(16):
        rowv = gg[sp]
        cp = (rowv + 255) & -256
        totp = totp + cp
        pmine = pmine + jnp.where((zeros16 + sp) < svec_b, cp, zeros16)
    qoffp = plsc.cumsum(totp) - totp
    startv = qoffp + pmine
    svec[...] = startv

    for d in range(16):
        startd = startv[d]

        def c_body(i, offv, d=d):
            ow = ownbuf[pl.ds(i * 16, 16)]
            pk = packbuf[pl.ds(i * 16, 16)]
            m = ow == d
            mi = m.astype(jnp.int32)
            excl = plsc.cumsum(mi) - mi
            plsc.store_scatter(outseg, [offv + excl], pk, mask=m)
            return offv + plsc.all_reduce_population_count(m)

        offv = lax.fori_loop(0, n16, c_body, zeros16)
        cnt = jnp.max(offv)
        trash16 = jnp.full((16,), TRASH, jnp.int32)
        for k in range(16):
            plsc.store_scatter(outseg, [cnt + k * 16 + iota], trash16)
        nch = ((cnt + 255) & -256) >> 8

        def dma_body(k, carry, startd=startd):
            pltpu.sync_copy(
                outseg.at[pl.ds(pl.multiple_of(k * 256, 256), 256)],
                qbuf.at[pl.ds(pl.multiple_of(startd + k * 256, 256), 256)])
            return carry

        lax.fori_loop(0, nch, dma_body, 0)
    plsc.subcore_barrier()

    qstart = jnp.sum(jnp.where(iota == svec_b, qoffp, zeros16))
    qlen = jnp.sum(jnp.where(iota == svec_b, totp, zeros16))
    nchunks = (qlen + CHQ - 1) // CHQ

    def region_body(g, carry0):
        for zb in range(8):
            pltpu.sync_copy(zb_sh, win.at[pl.ds(zb * 16, 16), :])

        def chunk_body(k, carry, g=g):
            pltpu.sync_copy(
                qbuf.at[pl.ds(pl.multiple_of(qstart + k * CHQ, 256), CHQ)],
                qstage)
            m16 = jnp.minimum(CHQ, qlen - k * CHQ) >> 4

            def j_body(j, plenv, g=g):
                w = qstage[pl.ds(j * 16, 16)]
                reg = w >> 27
                mt = reg == g
                r = (w >> 18) & 511
                gpid = (w & PIDM) + c * N
                mi = mt.astype(jnp.int32)
                excl = plsc.cumsum(mi) - mi
                slots = plenv + excl
                plsc.store_scatter(rlist, [slots], r, mask=mt)
                plsc.store_scatter(plist, [slots], gpid, mask=mt)
                return plenv + plsc.all_reduce_population_count(mt)

            plenv = lax.fori_loop(0, m16, j_body, zeros16)
            plen = jnp.max(plenv)
            plsc.store_scatter(rlist, [plen + iota],
                               jnp.full((16,), 512, jnp.int32))
            plsc.store_scatter(plist, [plen + iota], zeros16 + c * N)
            nblk = (plen + 15) >> 4

            def blk_body(bk, carry2):
                pltpu.async_copy(feat_hbm.at[plist.at[pl.ds(bk * 16, 16)]],
                                 fstage, sem).wait()
                r16 = rlist[pl.ds(bk * 16, 16)]
                for p in range(16):
                    rsp = zeros16 + r16[p]
                    mv = rsp < UROWS
                    for fc in range(8):
                        cur = plsc.load_gather(win, [fvecs[fc], rsp], mask=mv)
                        val = fstage[p, pl.ds(fc * 16, 16)]
                        plsc.store_scatter(win, [fvecs[fc], rsp],
                                           jnp.maximum(cur, val), mask=mv)
                return carry2

            lax.fori_loop(0, nblk, blk_body, 0)
            return carry

        lax.fori_loop(0, nchunks, chunk_body, 0)
        unit = g * 16 + s
        pltpu.sync_copy(
            win,
            out_hbm.at[c, :, pl.ds(pl.multiple_of(unit * UROWS, UROWS),
                                   UROWS)])
        return carry0

    lax.fori_loop(0, NREG, region_body, 0)


def kernel(points, W1, b1, g1, be1, W2, b2, g2, be2, W3, b3, g3, be3):
    sc = 1.0 / jnp.sqrt(1.0 + EPS)
    a1 = (W1.T * (g1 * sc)).astype(jnp.float32)
    c1 = (b1 * g1 * sc + be1)[None, :]
    a2 = (W2.T * (g2 * sc)).astype(jnp.float32)
    c2 = (b2 * g2 * sc + be2)[None, :]
    a3 = (W3.T * (g3 * sc)).astype(jnp.float32)
    c3 = (b3 * g3 * sc + be3)[None, :]
    pts = points.reshape(TOT, 4)
    feat, idx = _mlp(pts, a1, c1, a2, c2, a3, c3)
    idxp = jnp.pad(idx.reshape(-1), (0, PAD_PTS), constant_values=B * CELLS)
    fm = _scatter_sc(feat, idxp)
    return fm.reshape(B, FD, H, WG)


# speedup vs baseline: 2.5294x; 2.5294x over previous
import functools

import jax
import jax.numpy as jnp
from jax import lax
from jax.experimental import pallas as pl
from jax.experimental.pallas import tpu as pltpu
from jax.experimental.pallas import tpu_sc as plsc

B, N = 2, 100000
H, WG = 256, 256
FD = 128
EPS = 1e-5
TOT = B * N
CELLS = H * WG
BLK = 8000
NBLK = TOT // BLK

NT = 16
SHARD = 6256
SHARD_LAST = N - 15 * SHARD
PAD_PTS = 192
UROWS = 512
NREG = 8
WROWSP = 528
WWORDS = UROWS * FD
CHQ = 4096
QCAP = N + 256 * 256 + CHQ
GB = 64
TRASH = 8 << 27
PIDM = (1 << 18) - 1


def _mlp_body(pts_ref, a1_ref, c1_ref, a2_ref, c2_ref, a3_ref, c3_ref,
              feat_ref, idx_ref):
    x = pts_ref[...]
    h = jnp.maximum(jnp.dot(x, a1_ref[...], preferred_element_type=jnp.float32)
                    + c1_ref[...], 0.0)
    h = jnp.maximum(jnp.dot(h, a2_ref[...], preferred_element_type=jnp.float32)
                    + c2_ref[...], 0.0)
    h = jnp.maximum(jnp.dot(h, a3_ref[...], preferred_element_type=jnp.float32)
                    + c3_ref[...], 0.0)
    feat_ref[...] = h

    bid = pl.program_id(0)
    row = bid * BLK + jax.lax.broadcasted_iota(jnp.int32, (BLK, 1), 0)
    b = row // N
    xx = x[:, 0:1]
    yy = x[:, 1:2]
    xn = (xx + 50.0) * 0.01
    yn = (yy + 50.0) * 0.01
    valid = (xn >= 0) & (xn <= 1) & (yn >= 0) & (yn <= 1)
    gx = jnp.clip((xn * (WG - 1)).astype(jnp.int32), 0, WG - 1)
    gy = jnp.clip((yn * (H - 1)).astype(jnp.int32), 0, H - 1)
    flat = b * CELLS + gy * WG + gx
    flat = jnp.where(valid, flat, B * CELLS)
    idx_ref[...] = flat


def _mlp(pts, a1, c1, a2, c2, a3, c3):
    return pl.pallas_call(
        _mlp_body,
        grid=(NBLK,),
        in_specs=[
            pl.BlockSpec((BLK, 4), lambda i: (i, 0)),
            pl.BlockSpec((4, 64), lambda i: (0, 0)),
            pl.BlockSpec((1, 64), lambda i: (0, 0)),
            pl.BlockSpec((64, 128), lambda i: (0, 0)),
            pl.BlockSpec((1, 128), lambda i: (0, 0)),
            pl.BlockSpec((128, 128), lambda i: (0, 0)),
            pl.BlockSpec((1, 128), lambda i: (0, 0)),
        ],
        out_specs=[
            pl.BlockSpec((BLK, 128), lambda i: (i, 0)),
            pl.BlockSpec((BLK, 1), lambda i: (i, 0)),
        ],
        out_shape=[
            jax.ShapeDtypeStruct((TOT, 128), jnp.float32),
            jax.ShapeDtypeStruct((TOT, 1), jnp.int32),
        ],
    )(pts, a1, c1, a2, c2, a3, c3)


_sc_mesh = plsc.VectorSubcoreMesh(core_axis_name="c", subcore_axis_name="s")


@functools.partial(
    pl.kernel,
    out_type=jax.ShapeDtypeStruct((B, CELLS * FD), jnp.float32),
    mesh=_sc_mesh,
    scratch_types=[
        pltpu.VMEM((WROWSP * FD,), jnp.float32),
        pltpu.VMEM((SHARD,), jnp.int32),
        pltpu.VMEM((SHARD + 272,), jnp.int32),
        pltpu.VMEM((SHARD,), jnp.int32),
        pltpu.VMEM((SHARD,), jnp.int32),
        pltpu.VMEM((CHQ,), jnp.int32),
        pltpu.VMEM((GB, FD), jnp.float32),
        pltpu.VMEM((CHQ + 2 * GB,), jnp.int32),
        pltpu.VMEM((CHQ + 2 * GB,), jnp.int32),
        pltpu.VMEM((16, 16), jnp.int32),
        pltpu.VMEM((16,), jnp.int32),
        pltpu.VMEM((16,), jnp.int32),
        pltpu.VMEM((256,), jnp.int32),
        pltpu.VMEM_SHARED((16, 16), jnp.int32),
        pltpu.VMEM_SHARED((QCAP,), jnp.int32),
        pltpu.VMEM_SHARED((WWORDS,), jnp.float32),
        pltpu.SemaphoreType.DMA,
    ],
    compiler_params=pltpu.CompilerParams(needs_layout_passes=False),
)
def _scatter_sc(feat_hbm, idx_hbm, out_hbm, win, idxbuf, outseg,
                packbuf, ownbuf, qstage, fstage, rlist, plist, gg, svec,
                rbuf, hist, cnt_sh, qbuf, zb_sh, sem):
    c = lax.axis_index("c")
    s = lax.axis_index("s")
    iota = lax.broadcasted_iota(jnp.int32, (16,), 0)
    zeros16 = jnp.zeros((16,), jnp.int32)
    zerosf16 = jnp.zeros((16,), jnp.float32)
    ones16 = jnp.ones((16,), jnp.int32)
    fvecs = [iota + fc * 16 for fc in range(8)]

    @pl.when(s == 0)
    def _init_zb():
        def zz_body(k, carry):
            win[pl.ds(k * 16, 16)] = jnp.zeros((16,), jnp.float32)
            return carry
        lax.fori_loop(0, 128, zz_body, 0)
        for zr in range(32):
            pltpu.sync_copy(win.at[pl.ds(0, 2048)],
                            zb_sh.at[pl.ds(zr * 2048, 2048)])

    nmine = jnp.where(s < NT - 1, SHARD, SHARD_LAST)
    n16 = nmine // 16
    base = c * N + s * SHARD

    pltpu.sync_copy(idx_hbm.at[pl.ds(pl.multiple_of(base, 16), SHARD)], idxbuf)

    for k in range(16):
        hist[pl.ds(k * 16, 16)] = zeros16

    def p1_body(i, carry):
        v = idxbuf[pl.ds(i * 16, 16)]
        local = jnp.minimum(v - c * CELLS, CELLS)
        unit = local >> 9
        owner = unit & 15
        region = unit >> 4
        r = local & 511
        pid = s * SHARD + i * 16 + iota
        pk = (region << 27) | (r << 18) | pid
        packbuf[pl.ds(i * 16, 16)] = pk
        ownbuf[pl.ds(i * 16, 16)] = owner
        plsc.addupdate_scatter(hist, [owner * 16 + iota], ones16)
        return carry

    lax.fori_loop(0, n16, p1_body, 0)

    cvec = zeros16
    for d in range(16):
        cs = jnp.sum(hist[pl.ds(d * 16, 16)])
        cvec = jnp.where(iota == d, zeros16 + cs, cvec)
    svec[...] = cvec
    pltpu.sync_copy(svec, cnt_sh.at[s])
    plsc.subcore_barrier()

    pltpu.sync_copy(cnt_sh, gg)
    totp = zeros16
    pmine = zeros16
    svec_b = zeros16 + s
    for sp in range(16):
        rowv = gg[sp]
        cp = (rowv + 255) & -256
        totp = totp + cp
        pmine = pmine + jnp.where((zeros16 + sp) < svec_b, cp, zeros16)
    qoffp = plsc.cumsum(totp) - totp
    startv = qoffp + pmine
    svec[...] = startv

    for d in range(16):
        startd = startv[d]

        def c_body(i, offv, d=d):
            ow = ownbuf[pl.ds(i * 16, 16)]
            pk = packbuf[pl.ds(i * 16, 16)]
            m = ow == d
            mi = m.astype(jnp.int32)
            excl = plsc.cumsum(mi) - mi
            plsc.store_scatter(outseg, [offv + excl], pk, mask=m)
            return offv + plsc.all_reduce_population_count(m)

        offv = lax.fori_loop(0, n16, c_body, zeros16)
        cnt = offv[0]
        trash16 = jnp.full((16,), TRASH, jnp.int32)
        for k in range(16):
            plsc.store_scatter(outseg, [cnt + k * 16 + iota], trash16)
        nch = ((cnt + 255) & -256) >> 8

        def dma_body(k, carry, startd=startd):
            pltpu.sync_copy(
                outseg.at[pl.ds(pl.multiple_of(k * 256, 256), 256)],
                qbuf.at[pl.ds(pl.multiple_of(startd + k * 256, 256), 256)])
            return carry

        lax.fori_loop(0, nch, dma_body, 0)
    plsc.subcore_barrier()

    qstart = jnp.sum(jnp.where(iota == svec_b, qoffp, zeros16))
    qlen = jnp.sum(jnp.where(iota == svec_b, totp, zeros16))
    nchunks = (qlen + CHQ - 1) // CHQ

    def region_body(g, carry0):
        pltpu.sync_copy(zb_sh, win.at[pl.ds(0, WWORDS)])

        def chunk_body(k, carry, g=g):
            pltpu.sync_copy(
                qbuf.at[pl.ds(pl.multiple_of(qstart + k * CHQ, 256), CHQ)],
                qstage)
            m16 = jnp.minimum(CHQ, qlen - k * CHQ) >> 4

            def j_body(j, plenv, g=g):
                w = qstage[pl.ds(j * 16, 16)]
                reg = w >> 27
                mt = reg == g
                cnt = plsc.all_reduce_population_count(mt)

                @pl.when(cnt[0] > 0)
                def _match():
                    r = (w >> 18) & 511
                    gpid = (w & PIDM) + c * N
                    mi = mt.astype(jnp.int32)
                    excl = plsc.cumsum(mi) - mi
                    slots = plenv + excl
                    plsc.store_scatter(rlist, [slots], r, mask=mt)
                    plsc.store_scatter(plist, [slots], gpid, mask=mt)

                return plenv + cnt

            plenv = lax.fori_loop(0, m16, j_body, zeros16)
            plen = plenv[0]
            for k in range(GB // 16):
                plsc.store_scatter(rlist, [plen + k * 16 + iota],
                                   jnp.full((16,), 512, jnp.int32))
                plsc.store_scatter(plist, [plen + k * 16 + iota],
                                   zeros16 + c * N)
            nblk = (plen + GB - 1) // GB

            def blk_body(bk, carry2):
                pltpu.async_copy(feat_hbm.at[plist.at[pl.ds(bk * GB, GB)]],
                                 fstage, sem).wait()
                for q in range(GB // 16):
                    r16 = rlist[pl.ds(bk * GB + q * 16, 16)]
                    for p in range(16):
                        off = pl.multiple_of(r16[p] * FD, FD)
                        for fc in range(8):
                            cur = win[pl.ds(off + fc * 16, 16)]
                            val = fstage[q * 16 + p, pl.ds(fc * 16, 16)]
                            win[pl.ds(off + fc * 16, 16)] = (
                                jnp.maximum(cur, val))
                return carry2

            lax.fori_loop(0, nblk, blk_body, 0)
            return carry

        lax.fori_loop(0, nchunks, chunk_body, 0)
        unit = g * 16 + s
        pltpu.sync_copy(
            win.at[pl.ds(0, WWORDS)],
            out_hbm.at[c, pl.ds(pl.multiple_of(unit * WWORDS, WWORDS),
                                WWORDS)])
        return carry0

    lax.fori_loop(0, NREG, region_body, 0)


def kernel(points, W1, b1, g1, be1, W2, b2, g2, be2, W3, b3, g3, be3):
    sc = 1.0 / jnp.sqrt(1.0 + EPS)
    a1 = (W1.T * (g1 * sc)).astype(jnp.float32)
    c1 = (b1 * g1 * sc + be1)[None, :]
    a2 = (W2.T * (g2 * sc)).astype(jnp.float32)
    c2 = (b2 * g2 * sc + be2)[None, :]
    a3 = (W3.T * (g3 * sc)).astype(jnp.float32)
    c3 = (b3 * g3 * sc + be3)[None, :]
    pts = points.reshape(TOT, 4)
    feat, idx = _mlp(pts, a1, c1, a2, c2, a3, c3)
    idxp = jnp.pad(idx.reshape(-1), (0, PAD_PTS), constant_values=B * CELLS)
    fm = _scatter_sc(feat, idxp)
    return fm.reshape(B, H, WG, FD).transpose(0, 3, 1, 2)
